# baseline (speedup 1.0000x reference)
import functools

import jax
import jax.numpy as jnp
from jax import lax
from jax.experimental import pallas as pl
from jax.experimental.pallas import tpu as pltpu
from jax.experimental.pallas import tpu_sc as plsc

NY, NX, C, N, BS = 496, 432, 64, 40000, 4
NYNX = NY * NX
BASE = BS * NYNX
NTILES = 32
SEG = BASE // NTILES
SEG_V = SEG // 16
N2 = 40064
NHALF = N // 2
UF = 9
ROWS = 64
ROWS_T = 48
SEGMAX = ROWS * NX
XV = NX // 16

_MESH = plsc.VectorSubcoreMesh(core_axis_name="c", subcore_axis_name="s")
_PARAMS = pltpu.CompilerParams(needs_layout_passes=False)


@functools.partial(
    pl.pallas_call,
    out_specs=pl.BlockSpec(memory_space=pl.ANY),
    out_shape=jax.ShapeDtypeStruct((C * N2,), jnp.float32),
    scratch_shapes=[
        pltpu.VMEM((C, N2), jnp.float32),
        pltpu.SemaphoreType.DMA,
    ],
)
def _to_cmajor_tc(vf_ref, flat_ref, vft_v, sem):
    vft_v[:, :N] = vf_ref[...].T
    for c in range(C):
        pltpu.make_async_copy(vft_v.at[c],
                              flat_ref.at[pl.ds(c * N2, N2)], sem).start()
    for c in range(C):
        pltpu.make_async_copy(vft_v.at[c],
                              flat_ref.at[pl.ds(c * N2, N2)], sem).wait()


@functools.partial(
    pl.kernel,
    out_type=jax.ShapeDtypeStruct((BS * C * NYNX,), jnp.float32),
    mesh=_MESH,
    compiler_params=_PARAMS,
    scratch_types=[
        pltpu.VMEM((SEG,), jnp.int32),
        pltpu.VMEM((N,), jnp.float32),
        pltpu.VMEM((SEG,), jnp.float32),
        pltpu.VMEM((NHALF,), jnp.int32),
        pltpu.SemaphoreType.DMA,
    ],
)
def _fill_canvas(vft_hbm, flat_hbm, out_hbm, inv_v, col_v, stage_v, flat_v,
                 osem):
    wid = lax.axis_index("s") * 2 + lax.axis_index("c")
    b = wid // 8
    seg_lo = (wid % 8) * SEG
    lo = wid * SEG

    empty = jnp.full((16,), -1, jnp.int32)

    def fill(i, _):
        inv_v[pl.ds(i * 16, 16)] = empty
        return 0

    lax.fori_loop(0, SEG_V, fill, 0)

    lane = lax.iota(jnp.int32, 16)

    for half in (0, 1):
        pltpu.sync_copy(flat_hbm.at[pl.ds(half * NHALF, NHALF)], flat_v)

        def scan(i, _):
            base16 = flat_v[pl.ds(i * 16, 16)]
            loc = base16 - lo
            mask = (loc >= 0) & (loc < SEG)
            loc = jnp.where(mask, loc, 0)
            ids = lane + (i * 16 + half * NHALF)
            plsc.store_scatter(inv_v, [loc], ids, mask=mask)
            return 0

        lax.fori_loop(0, NHALF // 16, scan, 0)

    def out_copy(c):
        return pltpu.make_async_copy(
            stage_v, out_hbm.at[pl.ds((b * C + c) * NYNX + seg_lo, SEG)],
            osem)

    def chan(c, _):
        pltpu.sync_copy(vft_hbm.at[pl.ds(c * N2, N)], col_v)

        @pl.when(c > 0)
        def _():
            out_copy(c - 1).wait()

        def gat(j, _):
            for u in range(UF):
                off = (j * UF + u) * 16
                iv = inv_v[pl.ds(off, 16)]
                idx = jnp.maximum(iv, 0)
                mult = jnp.where(iv >= 0, jnp.float32(1.0), jnp.float32(0.0))
                g = plsc.load_gather(col_v, [idx])
                stage_v[pl.ds(off, 16)] = g * mult
            return 0

        lax.fori_loop(0, SEG_V // UF, gat, 0)
        out_copy(c).start()
        return 0

    lax.fori_loop(0, C, chan, 0)
    out_copy(C - 1).wait()


def kernel(voxel_features, coors, batch_size):
    flat = (coors[:, 0] * NYNX + coors[:, 2] * NX + coors[:, 3]).astype(jnp.int32)
    vft = _to_cmajor_tc(voxel_features)
    out = _fill_canvas(vft, flat)
    del batch_size
    return out.reshape(BS, C, NY, NX)


_ = (ROWS, ROWS_T, SEGMAX, XV)


# speedup vs baseline: 1.2983x; 1.2983x over previous
import functools

import jax
import jax.numpy as jnp
from jax import lax
from jax.experimental import pallas as pl
from jax.experimental.pallas import tpu as pltpu
from jax.experimental.pallas import tpu_sc as plsc

NY, NX, C, N, BS = 496, 432, 64, 40000, 4
NYNX = NY * NX
BASE = BS * NYNX
NTILES = 32
SEG = BASE // NTILES
SEG_V = SEG // 16
N2 = 40064
NHALF = N // 2
UF = 6
ROWS = 64
ROWS_T = 48
SEGMAX = ROWS * NX
XV = NX // 16

_MESH = plsc.VectorSubcoreMesh(core_axis_name="c", subcore_axis_name="s")
_PARAMS = pltpu.CompilerParams(needs_layout_passes=False)


@functools.partial(
    pl.pallas_call,
    out_specs=pl.BlockSpec(memory_space=pl.ANY),
    out_shape=jax.ShapeDtypeStruct((C * N2,), jnp.float32),
    scratch_shapes=[
        pltpu.VMEM((C, N2), jnp.float32),
        pltpu.SemaphoreType.DMA,
    ],
)
def _to_cmajor_tc(vf_ref, flat_ref, vft_v, sem):
    vft_v[:, :N] = vf_ref[...].T
    for c in range(C):
        pltpu.make_async_copy(vft_v.at[c],
                              flat_ref.at[pl.ds(c * N2, N2)], sem).start()
    for c in range(C):
        pltpu.make_async_copy(vft_v.at[c],
                              flat_ref.at[pl.ds(c * N2, N2)], sem).wait()


@functools.partial(
    pl.kernel,
    out_type=jax.ShapeDtypeStruct((BS * C * NYNX,), jnp.float32),
    mesh=_MESH,
    compiler_params=_PARAMS,
    scratch_types=[
        pltpu.VMEM((SEG,), jnp.int32),
        pltpu.VMEM((N,), jnp.float32),
        pltpu.VMEM((SEG,), jnp.float32),
        pltpu.VMEM((NHALF,), jnp.int32),
        pltpu.SemaphoreType.DMA,
    ],
)
def _fill_canvas(vft_hbm, flat_hbm, out_hbm, inv_v, col_v, stage_v, flat_v,
                 osem):
    wid = lax.axis_index("s") * 2 + lax.axis_index("c")
    b = wid // 8
    seg_lo = (wid % 8) * SEG
    lo = wid * SEG

    empty = jnp.full((16,), -1, jnp.int32)

    def fill(i, _):
        inv_v[pl.ds(i * 16, 16)] = empty
        return 0

    lax.fori_loop(0, SEG_V, fill, 0)

    lane = lax.iota(jnp.int32, 16)

    for half in (0, 1):
        pltpu.sync_copy(flat_hbm.at[pl.ds(half * NHALF, NHALF)], flat_v)

        def scan(i, _):
            base16 = flat_v[pl.ds(i * 16, 16)]
            loc = base16 - lo
            mask = (loc >= 0) & (loc < SEG)
            loc = jnp.where(mask, loc, 0)
            ids = lane + (i * 16 + half * NHALF)
            plsc.store_scatter(inv_v, [loc], ids, mask=mask)
            return 0

        lax.fori_loop(0, NHALF // 16, scan, 0)

    def out_copy(c):
        return pltpu.make_async_copy(
            stage_v, out_hbm.at[pl.ds((b * C + c) * NYNX + seg_lo, SEG)],
            osem)

    def chan(c, _):
        pltpu.sync_copy(vft_hbm.at[pl.ds(c * N2, N)], col_v)

        @pl.when(c > 0)
        def _():
            out_copy(c - 1).wait()

        def gat(j, _):
            for u in range(UF):
                off = (j * UF + u) * 16
                iv = inv_v[pl.ds(off, 16)]
                idx = jnp.maximum(iv, 0)
                mult = jnp.where(iv >= 0, jnp.float32(1.0), jnp.float32(0.0))
                g = plsc.load_gather(col_v, [idx])
                stage_v[pl.ds(off, 16)] = g * mult
            return 0

        lax.fori_loop(0, SEG_V // UF, gat, 0)
        out_copy(c).start()
        return 0

    lax.fori_loop(0, C, chan, 0)
    out_copy(C - 1).wait()


def kernel(voxel_features, coors, batch_size):
    flat = (coors[:, 0] * NYNX + coors[:, 2] * NX + coors[:, 3]).astype(jnp.int32)
    vft = _to_cmajor_tc(voxel_features)
    out = _fill_canvas(vft, flat)
    del batch_size
    return out.reshape(BS, C, NY, NX)


_ = (ROWS, ROWS_T, SEGMAX, XV)
